# baseline (speedup 1.0000x reference)
import functools

import jax
import jax.numpy as jnp
from jax import lax
from jax.experimental import pallas as pl
from jax.experimental.pallas import tpu as pltpu
from jax.experimental.pallas import tpu_sc as plsc

N = 50000
E = 800000
G = 64
D1 = 32

NC, NS, L = 2, 16, 16
NW = NC * NS

CHUNK = 128
NCW = 200
KC = 20
NG = NCW // KC
E_PAD = NW * NCW * CHUNK
ROWS_PER_TILE = 3200
N_ACC = NS * ROWS_PER_TILE

PR = 1568
N_POOL = NW * PR
GP = G + 1

BLK = 2000
NBLK = N // BLK


def _make_sc_agg(d):
    mesh = plsc.VectorSubcoreMesh(
        core_axis_name="c", subcore_axis_name="s", num_cores=NC, num_subcores=NS
    )

    @functools.partial(
        pl.kernel,
        out_type=jax.ShapeDtypeStruct((NC, N_ACC, d), jnp.float32),
        mesh=mesh,
        compiler_params=pltpu.CompilerParams(use_tc_tiling_on_sc=False),
        scratch_types=[
            pltpu.VMEM((KC, CHUNK), jnp.int32),
            pltpu.VMEM((KC, CHUNK), jnp.int32),
            pltpu.VMEM((KC, CHUNK), jnp.int32),
            pltpu.VMEM((KC, CHUNK), jnp.int32),
            pltpu.VMEM((CHUNK, d), jnp.float32),
            pltpu.VMEM((CHUNK, d), jnp.float32),
            pltpu.VMEM((CHUNK, d), jnp.float32),
            pltpu.VMEM((CHUNK, d), jnp.float32),
            pltpu.VMEM_SHARED((N_ACC, d), jnp.float32),
            pltpu.SemaphoreType.DMA,
            pltpu.SemaphoreType.DMA,
            pltpu.SemaphoreType.DMA,
            pltpu.SemaphoreType.DMA,
            pltpu.SemaphoreType.DMA,
            pltpu.SemaphoreType.DMA,
            pltpu.SemaphoreType.DMA,
            pltpu.SemaphoreType.DMA,
            pltpu.SemaphoreType.DMA,
        ],
    )
    def agg(h_hbm, src_hbm, dst_hbm, zer_hbm, out_hbm,
            src_vA, dst_vA, src_vB, dst_vB, b0, b1, b2, b3, acc,
            g0s, g1s, g2s, g3s, s0s, s1s, s2s, s3s, isem):
        cid = lax.axis_index("c")
        sid = lax.axis_index("s")
        wid = cid * NS + sid
        bufs = (b0, b1, b2, b3)
        gsems = (g0s, g1s, g2s, g3s)
        ssems = (s0s, s1s, s2s, s3s)

        pltpu.sync_copy(zer_hbm, acc.at[pl.ds(sid * ROWS_PER_TILE, ROWS_PER_TILE), :])
        plsc.subcore_barrier()

        def process_chunks(src_v, dst_v):
            for b in range(4):
                pltpu.async_copy(h_hbm.at[src_v.at[b]], bufs[b], gsems[b])

            def qbody(t, _):
                jb = 4 * t
                for b in range(4):
                    j = jb + b
                    pltpu.make_async_copy(
                        h_hbm.at[src_v.at[j]], bufs[b], gsems[b]
                    ).wait()
                    pltpu.async_copy(
                        bufs[b], acc.at[dst_v.at[j]], ssems[b], add=True
                    )
                for b in range(4):
                    nj = jb + b + 4
                    @pl.when(nj < KC)
                    def _(b=b, nj=nj, jb=jb):
                        pltpu.make_async_copy(
                            bufs[b], acc.at[dst_v.at[jb + b]], ssems[b]
                        ).wait()
                        pltpu.async_copy(h_hbm.at[src_v.at[nj]], bufs[b], gsems[b])
                return 0

            lax.fori_loop(0, KC // 4, qbody, 0)
            for b in range(4):
                pltpu.make_async_copy(
                    bufs[b], acc.at[dst_v.at[KC - 4 + b]], ssems[b]
                ).wait()

        def stage(g, sv, dv, sem):
            pltpu.async_copy(src_hbm.at[wid, pl.ds(g * KC, KC)], sv, sem)
            pltpu.async_copy(dst_hbm.at[wid, pl.ds(g * KC, KC)], dv, sem)

        def stage_wait(g, sv, dv, sem):
            pltpu.make_async_copy(src_hbm.at[wid, pl.ds(g * KC, KC)], sv, sem).wait()
            pltpu.make_async_copy(dst_hbm.at[wid, pl.ds(g * KC, KC)], dv, sem).wait()

        pltpu.sync_copy(src_hbm.at[wid, pl.ds(0, KC)], src_vA)
        pltpu.sync_copy(dst_hbm.at[wid, pl.ds(0, KC)], dst_vA)

        def gpair(q, _):
            ga = 2 * q
            stage(ga + 1, src_vB, dst_vB, isem)
            process_chunks(src_vA, dst_vA)
            stage_wait(ga + 1, src_vB, dst_vB, isem)

            @pl.when(ga + 2 < NG)
            def _():
                stage(ga + 2, src_vA, dst_vA, isem)

            process_chunks(src_vB, dst_vB)

            @pl.when(ga + 2 < NG)
            def _():
                stage_wait(ga + 2, src_vA, dst_vA, isem)

            return 0

        lax.fori_loop(0, NG // 2, gpair, 0)
        plsc.subcore_barrier()

        pltpu.sync_copy(
            acc.at[pl.ds(sid * ROWS_PER_TILE, ROWS_PER_TILE), :],
            out_hbm.at[cid, pl.ds(sid * ROWS_PER_TILE, ROWS_PER_TILE), :],
        )

    return agg


_sc_agg8 = _make_sc_agg(8)
_sc_agg32 = _make_sc_agg(32)


def _make_sc_pool():
    mesh = plsc.VectorSubcoreMesh(
        core_axis_name="c", subcore_axis_name="s", num_cores=NC, num_subcores=NS
    )

    @functools.partial(
        pl.kernel,
        out_type=jax.ShapeDtypeStruct((NW, GP, D1), jnp.float32),
        mesh=mesh,
        compiler_params=pltpu.CompilerParams(use_tc_tiling_on_sc=False),
        scratch_types=[
            pltpu.VMEM((PR,), jnp.int32),
            pltpu.VMEM((PR, D1), jnp.float32),
            pltpu.VMEM((GP, D1), jnp.float32),
        ],
    )
    def pool(h_hbm, b_hbm, out_hbm, batch_v, h_v, acc):
        cid = lax.axis_index("c")
        sid = lax.axis_index("s")
        wid = cid * NS + sid
        base = wid * PR
        pltpu.sync_copy(b_hbm.at[pl.ds(base, PR)], batch_v)
        pltpu.sync_copy(h_hbm.at[pl.ds(base, PR), :], h_v)

        neg = jnp.full((L,), -3.4e38, jnp.float32)

        def ini(g, _):
            acc[g, pl.ds(0, L)] = neg
            acc[g, pl.ds(L, L)] = neg
            return 0

        lax.fori_loop(0, GP, ini, 0)

        def body(q, _):
            gvec = batch_v[pl.ds(q * L, L)]
            for lane in range(L):
                r = q * L + lane
                g = gvec[lane]
                acc[g, pl.ds(0, L)] = jnp.maximum(
                    acc[g, pl.ds(0, L)], h_v[r, pl.ds(0, L)]
                )
                acc[g, pl.ds(L, L)] = jnp.maximum(
                    acc[g, pl.ds(L, L)], h_v[r, pl.ds(L, L)]
                )
            return 0

        lax.fori_loop(0, PR // L, body, 0)
        pltpu.sync_copy(acc, out_hbm.at[wid])

    return pool


_sc_pool = _make_sc_pool()


def _make_layer(d):
    def body(h_ref, agg_ref, w1, b1, w2, b2, t_ref, st_ref, accs):
        i = pl.program_id(0)
        u = h_ref[...] + agg_ref[0] + agg_ref[1]
        t = jnp.dot(u, w1[...], preferred_element_type=jnp.float32) + b1[...]
        t = jnp.maximum(t, 0.0)
        t = jnp.dot(t, w2[...], preferred_element_type=jnp.float32) + b2[...]
        t = jnp.maximum(t, 0.0)
        t_ref[...] = t

        @pl.when(i == 0)
        def _():
            accs[...] = jnp.zeros_like(accs)

        accs[0:1, :] += jnp.sum(t, axis=0, keepdims=True)
        accs[1:2, :] += jnp.sum(t * t, axis=0, keepdims=True)

        @pl.when(i == NBLK - 1)
        def _():
            st_ref[...] = accs[...]

    return pl.pallas_call(
        body,
        grid=(NBLK,),
        in_specs=[
            pl.BlockSpec((BLK, d), lambda i: (i, 0)),
            pl.BlockSpec((2, BLK, d), lambda i: (0, i, 0)),
            pl.BlockSpec((d, D1), lambda i: (0, 0)),
            pl.BlockSpec((1, D1), lambda i: (0, 0)),
            pl.BlockSpec((D1, D1), lambda i: (0, 0)),
            pl.BlockSpec((1, D1), lambda i: (0, 0)),
        ],
        out_specs=[
            pl.BlockSpec((BLK, D1), lambda i: (i, 0)),
            pl.BlockSpec((8, D1), lambda i: (0, 0)),
        ],
        out_shape=[
            jax.ShapeDtypeStruct((N, D1), jnp.float32),
            jax.ShapeDtypeStruct((8, D1), jnp.float32),
        ],
        scratch_shapes=[pltpu.VMEM((8, D1), jnp.float32)],
    )


_layer8 = _make_layer(8)
_layer32 = _make_layer(32)


def _make_norm(nrows_out):
    def body(t_ref, s_ref, c_ref, o_ref):
        o_ref[...] = t_ref[...] * s_ref[...] + c_ref[...]

    return pl.pallas_call(
        body,
        grid=(NBLK,),
        in_specs=[
            pl.BlockSpec((BLK, D1), lambda i: (i, 0)),
            pl.BlockSpec((1, D1), lambda i: (0, 0)),
            pl.BlockSpec((1, D1), lambda i: (0, 0)),
        ],
        out_specs=pl.BlockSpec((BLK, D1), lambda i: (i, 0)),
        out_shape=jax.ShapeDtypeStruct((nrows_out, D1), jnp.float32),
    )


_norm_n = _make_norm(N)
_norm_pool = _make_norm(N_POOL)


def _head(pmb, pmr, wb, bb, wr, br2, wbe, bbe, wm, bm):
    def body(pmb_ref, pmr_ref, wb_r, bb_r, wr_r, br_r, wbe_r, bbe_r, wm_r, bm_r, o_ref):
        pb = jnp.max(pmb_ref[...], axis=0)[:G, :]
        eb = jnp.maximum(
            jnp.dot(pb, wb_r[...], preferred_element_type=jnp.float32) + bb_r[...], 0.0
        )
        pr = jnp.max(pmr_ref[...], axis=0)[:G, :]
        er = jnp.maximum(
            jnp.dot(pr, wr_r[...], preferred_element_type=jnp.float32) + br_r[...], 0.0
        )
        cat = jnp.concatenate([eb, er], axis=-1)
        h = jnp.maximum(
            jnp.dot(cat, wbe_r[...], preferred_element_type=jnp.float32) + bbe_r[...], 0.0
        )
        z = jnp.dot(h, wm_r[...], preferred_element_type=jnp.float32) + bm_r[...]
        o_ref[...] = 1.0 / (1.0 + jnp.exp(-z))

    return pl.pallas_call(
        body,
        out_shape=jax.ShapeDtypeStruct((G, 1), jnp.float32),
    )(pmb, pmr, wb, bb, wr, br2, wbe, bbe, wm, bm)


def _prep_edges(ei):
    src = jnp.concatenate([ei[0], jnp.zeros((E_PAD - E,), jnp.int32)])
    pad_dst = N + (jnp.arange(E_PAD - E, dtype=jnp.int32) % (N_ACC - N))
    dst = jnp.concatenate([ei[1], pad_dst])
    return src.reshape(NW, NCW, CHUNK), dst.reshape(NW, NCW, CHUNK)


def _branch(x, ei, batch, br, p):
    srcr, dstr = _prep_edges(ei)
    batch_p = jnp.pad(batch, (0, N_POOL - N), constant_values=G).astype(jnp.int32)
    zer8 = jnp.zeros((ROWS_PER_TILE, 8), jnp.float32)
    zer32 = jnp.zeros((ROWS_PER_TILE, D1), jnp.float32)

    h = jnp.pad(x, ((0, 0), (0, 8 - x.shape[1])))
    for i in range(1, 4):
        d = 8 if i == 1 else D1
        w1 = p[br + "_c%d_W1" % i]
        if i == 1:
            w1 = jnp.pad(w1, ((0, 8 - w1.shape[0]), (0, 0)))
        agg = (_sc_agg8 if d == 8 else _sc_agg32)(
            h, srcr, dstr, zer8 if d == 8 else zer32
        )
        t, st = (_layer8 if d == 8 else _layer32)(
            h, agg, w1,
            p[br + "_c%d_b1" % i].reshape(1, D1),
            p[br + "_c%d_W2" % i],
            p[br + "_c%d_b2" % i].reshape(1, D1),
        )
        mean = st[0] / N
        var = st[1] / N - mean * mean
        s = p[br + "_bn%d_g" % i] * lax.rsqrt(var + 1e-5)
        c = p[br + "_bn%d_b" % i] - mean * s
        norm = _norm_n if i < 3 else _norm_pool
        h = norm(t, s.reshape(1, D1), c.reshape(1, D1))

    return _sc_pool(h, batch_p)


def kernel(data_base, edge_index_base, batch_base,
           data_residual, edge_index_residual, batch_residual, params):
    p = params
    pmb = _branch(data_base, edge_index_base, batch_base, "base", p)
    pmr = _branch(data_residual, edge_index_residual, batch_residual, "res", p)
    return _head(
        pmb, pmr,
        p["base_Wbr"], p["base_bbr"].reshape(1, D1),
        p["res_Wbr"], p["res_bbr"].reshape(1, D1),
        p["W_before"], p["b_before"].reshape(1, 16),
        p["W_mean"], p["b_mean"].reshape(1, 1),
    )


# speedup vs baseline: 1.0098x; 1.0098x over previous
import functools

import jax
import jax.numpy as jnp
from jax import lax
from jax.experimental import pallas as pl
from jax.experimental.pallas import tpu as pltpu
from jax.experimental.pallas import tpu_sc as plsc

N = 50000
E = 800000
G = 64
D1 = 32

NC, NS, L = 2, 16, 16
NW = NC * NS

CHUNK = 128
NCW = 400
KC = 20
TOT_CH = NS * NCW
E_PAD = TOT_CH * CHUNK
FAST_CID = 0
ROWS_PER_TILE = 3200
N_ACC = NS * ROWS_PER_TILE

PR = 1568
N_POOL = NW * PR
GP = G + 1

BLK = 2000
NBLK = N // BLK


def _make_sc_agg(d, ncw_f, ncw_s):
    ng_f, ng_s = ncw_f // KC, ncw_s // KC
    mesh = plsc.VectorSubcoreMesh(
        core_axis_name="c", subcore_axis_name="s", num_cores=NC, num_subcores=NS
    )

    @functools.partial(
        pl.kernel,
        out_type=jax.ShapeDtypeStruct((NC, N_ACC, d), jnp.float32),
        mesh=mesh,
        compiler_params=pltpu.CompilerParams(use_tc_tiling_on_sc=False),
        scratch_types=[
            pltpu.VMEM((KC, CHUNK), jnp.int32),
            pltpu.VMEM((KC, CHUNK), jnp.int32),
            pltpu.VMEM((KC, CHUNK), jnp.int32),
            pltpu.VMEM((KC, CHUNK), jnp.int32),
            pltpu.VMEM((CHUNK, d), jnp.float32),
            pltpu.VMEM((CHUNK, d), jnp.float32),
            pltpu.VMEM((CHUNK, d), jnp.float32),
            pltpu.VMEM((CHUNK, d), jnp.float32),
            pltpu.VMEM_SHARED((N_ACC, d), jnp.float32),
            pltpu.SemaphoreType.DMA,
            pltpu.SemaphoreType.DMA,
            pltpu.SemaphoreType.DMA,
            pltpu.SemaphoreType.DMA,
            pltpu.SemaphoreType.DMA,
            pltpu.SemaphoreType.DMA,
            pltpu.SemaphoreType.DMA,
            pltpu.SemaphoreType.DMA,
            pltpu.SemaphoreType.DMA,
        ],
    )
    def agg(h_hbm, src_hbm, dst_hbm, zer_hbm, out_hbm,
            src_vA, dst_vA, src_vB, dst_vB, b0, b1, b2, b3, acc,
            g0s, g1s, g2s, g3s, s0s, s1s, s2s, s3s, isem):
        cid = lax.axis_index("c")
        sid = lax.axis_index("s")
        is_fast = cid == FAST_CID
        cbase = jnp.where(is_fast, sid * ncw_f, NS * ncw_f + sid * ncw_s)
        npair = jnp.where(is_fast, ng_f // 2, ng_s // 2)
        bufs = (b0, b1, b2, b3)
        gsems = (g0s, g1s, g2s, g3s)
        ssems = (s0s, s1s, s2s, s3s)

        pltpu.sync_copy(zer_hbm, acc.at[pl.ds(sid * ROWS_PER_TILE, ROWS_PER_TILE), :])
        plsc.subcore_barrier()

        def process_chunks(src_v, dst_v):
            for b in range(4):
                pltpu.async_copy(h_hbm.at[src_v.at[b]], bufs[b], gsems[b])

            def qbody(t, _):
                jb = 4 * t
                for b in range(4):
                    j = jb + b
                    pltpu.make_async_copy(
                        h_hbm.at[src_v.at[j]], bufs[b], gsems[b]
                    ).wait()
                    pltpu.async_copy(
                        bufs[b], acc.at[dst_v.at[j]], ssems[b], add=True
                    )
                for b in range(4):
                    nj = jb + b + 4
                    @pl.when(nj < KC)
                    def _(b=b, nj=nj, jb=jb):
                        pltpu.make_async_copy(
                            bufs[b], acc.at[dst_v.at[jb + b]], ssems[b]
                        ).wait()
                        pltpu.async_copy(h_hbm.at[src_v.at[nj]], bufs[b], gsems[b])
                return 0

            lax.fori_loop(0, KC // 4, qbody, 0)
            for b in range(4):
                pltpu.make_async_copy(
                    bufs[b], acc.at[dst_v.at[KC - 4 + b]], ssems[b]
                ).wait()

        def stage(g, sv, dv, sem):
            pltpu.async_copy(src_hbm.at[pl.ds(cbase + g * KC, KC)], sv, sem)
            pltpu.async_copy(dst_hbm.at[pl.ds(cbase + g * KC, KC)], dv, sem)

        def stage_wait(g, sv, dv, sem):
            pltpu.make_async_copy(
                src_hbm.at[pl.ds(cbase + g * KC, KC)], sv, sem
            ).wait()
            pltpu.make_async_copy(
                dst_hbm.at[pl.ds(cbase + g * KC, KC)], dv, sem
            ).wait()

        ng = 2 * npair
        pltpu.sync_copy(src_hbm.at[pl.ds(cbase, KC)], src_vA)
        pltpu.sync_copy(dst_hbm.at[pl.ds(cbase, KC)], dst_vA)

        def gpair(q, _):
            ga = 2 * q
            stage(ga + 1, src_vB, dst_vB, isem)
            process_chunks(src_vA, dst_vA)
            stage_wait(ga + 1, src_vB, dst_vB, isem)

            @pl.when(ga + 2 < ng)
            def _():
                stage(ga + 2, src_vA, dst_vA, isem)

            process_chunks(src_vB, dst_vB)

            @pl.when(ga + 2 < ng)
            def _():
                stage_wait(ga + 2, src_vA, dst_vA, isem)

            return 0

        lax.fori_loop(0, npair, gpair, 0)
        plsc.subcore_barrier()

        pltpu.sync_copy(
            acc.at[pl.ds(sid * ROWS_PER_TILE, ROWS_PER_TILE), :],
            out_hbm.at[cid, pl.ds(sid * ROWS_PER_TILE, ROWS_PER_TILE), :],
        )

    return agg


_sc_agg8 = _make_sc_agg(8, 240, 160)
_sc_agg32 = _make_sc_agg(32, 280, 120)


def _make_sc_pool():
    mesh = plsc.VectorSubcoreMesh(
        core_axis_name="c", subcore_axis_name="s", num_cores=NC, num_subcores=NS
    )

    @functools.partial(
        pl.kernel,
        out_type=jax.ShapeDtypeStruct((NW, GP, D1), jnp.float32),
        mesh=mesh,
        compiler_params=pltpu.CompilerParams(use_tc_tiling_on_sc=False),
        scratch_types=[
            pltpu.VMEM((PR,), jnp.int32),
            pltpu.VMEM((PR, D1), jnp.float32),
            pltpu.VMEM((GP, D1), jnp.float32),
        ],
    )
    def pool(h_hbm, b_hbm, out_hbm, batch_v, h_v, acc):
        cid = lax.axis_index("c")
        sid = lax.axis_index("s")
        wid = cid * NS + sid
        base = wid * PR
        pltpu.sync_copy(b_hbm.at[pl.ds(base, PR)], batch_v)
        pltpu.sync_copy(h_hbm.at[pl.ds(base, PR), :], h_v)

        neg = jnp.full((L,), -3.4e38, jnp.float32)

        def ini(g, _):
            acc[g, pl.ds(0, L)] = neg
            acc[g, pl.ds(L, L)] = neg
            return 0

        lax.fori_loop(0, GP, ini, 0)

        def body(q, _):
            gvec = batch_v[pl.ds(q * L, L)]
            for lane in range(L):
                r = q * L + lane
                g = gvec[lane]
                acc[g, pl.ds(0, L)] = jnp.maximum(
                    acc[g, pl.ds(0, L)], h_v[r, pl.ds(0, L)]
                )
                acc[g, pl.ds(L, L)] = jnp.maximum(
                    acc[g, pl.ds(L, L)], h_v[r, pl.ds(L, L)]
                )
            return 0

        lax.fori_loop(0, PR // L, body, 0)
        pltpu.sync_copy(acc, out_hbm.at[wid])

    return pool


_sc_pool = _make_sc_pool()


def _make_layer(d):
    def body(h_ref, agg_ref, w1, b1, w2, b2, t_ref, st_ref, accs):
        i = pl.program_id(0)
        u = h_ref[...] + agg_ref[0] + agg_ref[1]
        t = jnp.dot(u, w1[...], preferred_element_type=jnp.float32) + b1[...]
        t = jnp.maximum(t, 0.0)
        t = jnp.dot(t, w2[...], preferred_element_type=jnp.float32) + b2[...]
        t = jnp.maximum(t, 0.0)
        t_ref[...] = t

        @pl.when(i == 0)
        def _():
            accs[...] = jnp.zeros_like(accs)

        s = jnp.sum(t, axis=0, keepdims=True)
        m = s * (1.0 / BLK)
        d2 = t - m
        accs[0:1, :] += s
        accs[1:2, :] += jnp.sum(d2 * d2, axis=0, keepdims=True)
        accs[2:3, :] += m * m

        @pl.when(i == NBLK - 1)
        def _():
            st_ref[...] = accs[...]

    return pl.pallas_call(
        body,
        grid=(NBLK,),
        in_specs=[
            pl.BlockSpec((BLK, d), lambda i: (i, 0)),
            pl.BlockSpec((2, BLK, d), lambda i: (0, i, 0)),
            pl.BlockSpec((d, D1), lambda i: (0, 0)),
            pl.BlockSpec((1, D1), lambda i: (0, 0)),
            pl.BlockSpec((D1, D1), lambda i: (0, 0)),
            pl.BlockSpec((1, D1), lambda i: (0, 0)),
        ],
        out_specs=[
            pl.BlockSpec((BLK, D1), lambda i: (i, 0)),
            pl.BlockSpec((8, D1), lambda i: (0, 0)),
        ],
        out_shape=[
            jax.ShapeDtypeStruct((N, D1), jnp.float32),
            jax.ShapeDtypeStruct((8, D1), jnp.float32),
        ],
        scratch_shapes=[pltpu.VMEM((8, D1), jnp.float32)],
    )


_layer8 = _make_layer(8)
_layer32 = _make_layer(32)


def _make_norm(nrows_out):
    def body(t_ref, s_ref, c_ref, o_ref):
        o_ref[...] = t_ref[...] * s_ref[...] + c_ref[...]

    return pl.pallas_call(
        body,
        grid=(NBLK,),
        in_specs=[
            pl.BlockSpec((BLK, D1), lambda i: (i, 0)),
            pl.BlockSpec((1, D1), lambda i: (0, 0)),
            pl.BlockSpec((1, D1), lambda i: (0, 0)),
        ],
        out_specs=pl.BlockSpec((BLK, D1), lambda i: (i, 0)),
        out_shape=jax.ShapeDtypeStruct((nrows_out, D1), jnp.float32),
    )


_norm_n = _make_norm(N)
_norm_pool = _make_norm(N_POOL)


def _head(pmb, pmr, wb, bb, wr, br2, wbe, bbe, wm, bm):
    def body(pmb_ref, pmr_ref, wb_r, bb_r, wr_r, br_r, wbe_r, bbe_r, wm_r, bm_r, o_ref):
        pb = jnp.max(pmb_ref[...], axis=0)[:G, :]
        eb = jnp.maximum(
            jnp.dot(pb, wb_r[...], preferred_element_type=jnp.float32) + bb_r[...], 0.0
        )
        pr = jnp.max(pmr_ref[...], axis=0)[:G, :]
        er = jnp.maximum(
            jnp.dot(pr, wr_r[...], preferred_element_type=jnp.float32) + br_r[...], 0.0
        )
        cat = jnp.concatenate([eb, er], axis=-1)
        h = jnp.maximum(
            jnp.dot(cat, wbe_r[...], preferred_element_type=jnp.float32) + bbe_r[...], 0.0
        )
        z = jnp.dot(h, wm_r[...], preferred_element_type=jnp.float32) + bm_r[...]
        o_ref[...] = 1.0 / (1.0 + jnp.exp(-z))

    return pl.pallas_call(
        body,
        out_shape=jax.ShapeDtypeStruct((G, 1), jnp.float32),
    )(pmb, pmr, wb, bb, wr, br2, wbe, bbe, wm, bm)


def _prep_edges(ei):
    src = jnp.concatenate([ei[0], jnp.zeros((E_PAD - E,), jnp.int32)])
    pad_dst = N + (jnp.arange(E_PAD - E, dtype=jnp.int32) % (N_ACC - N))
    dst = jnp.concatenate([ei[1], pad_dst])
    return src.reshape(TOT_CH, CHUNK), dst.reshape(TOT_CH, CHUNK)


def _branch(x, ei, batch, br, p):
    srcr, dstr = _prep_edges(ei)
    batch_p = jnp.pad(batch, (0, N_POOL - N), constant_values=G).astype(jnp.int32)
    zer8 = jnp.zeros((ROWS_PER_TILE, 8), jnp.float32)
    zer32 = jnp.zeros((ROWS_PER_TILE, D1), jnp.float32)

    h = jnp.pad(x, ((0, 0), (0, 8 - x.shape[1])))
    for i in range(1, 4):
        d = 8 if i == 1 else D1
        w1 = p[br + "_c%d_W1" % i]
        if i == 1:
            w1 = jnp.pad(w1, ((0, 8 - w1.shape[0]), (0, 0)))
        agg = (_sc_agg8 if d == 8 else _sc_agg32)(
            h, srcr, dstr, zer8 if d == 8 else zer32
        )
        t, st = (_layer8 if d == 8 else _layer32)(
            h, agg, w1,
            p[br + "_c%d_b1" % i].reshape(1, D1),
            p[br + "_c%d_W2" % i],
            p[br + "_c%d_b2" % i].reshape(1, D1),
        )
        mean = st[0] / N
        var = st[1] / N + (st[2] / NBLK - mean * mean)
        s = p[br + "_bn%d_g" % i] * lax.rsqrt(var + 1e-5)
        c = p[br + "_bn%d_b" % i] - mean * s
        norm = _norm_n if i < 3 else _norm_pool
        h = norm(t, s.reshape(1, D1), c.reshape(1, D1))

    return _sc_pool(h, batch_p)


def kernel(data_base, edge_index_base, batch_base,
           data_residual, edge_index_residual, batch_residual, params):
    p = params
    pmb = _branch(data_base, edge_index_base, batch_base, "base", p)
    pmr = _branch(data_residual, edge_index_residual, batch_residual, "res", p)
    return _head(
        pmb, pmr,
        p["base_Wbr"], p["base_bbr"].reshape(1, D1),
        p["res_Wbr"], p["res_bbr"].reshape(1, D1),
        p["W_before"], p["b_before"].reshape(1, 16),
        p["W_mean"], p["b_mean"].reshape(1, 1),
    )
